# Initial kernel scaffold; baseline (speedup 1.0000x reference)
import jax
import jax.numpy as jnp
from jax.experimental import pallas as pl


def kernel(x, table):
    raise NotImplementedError("write your pallas kernel here")


# speedup vs baseline: 1.0053x; 1.0053x over previous
import functools
import math

import jax
import jax.numpy as jnp
import numpy as np
from jax import lax
from jax.experimental import pallas as pl
from jax.experimental.pallas import tpu as pltpu
from jax.experimental.pallas import tpu_sc as plsc

VOCAB = 100000
D_MODEL = 768
MAX_POS = 2048
SCALE = math.sqrt(float(D_MODEL))


def _positional_encoding(length, depth):
    depth_h = depth / 2
    positions = np.arange(length)[:, np.newaxis]
    depths = np.arange(depth_h)[np.newaxis, :] / depth_h
    angle_rates = 1 / 10000 ** depths
    angle_rads = positions * angle_rates
    return np.concatenate(
        [np.sin(angle_rads), np.cos(angle_rads)], axis=-1
    ).astype(np.float32)


_POS_ENC = _positional_encoding(MAX_POS, D_MODEL)

NC, NS = 2, 16
NW = NC * NS
LANES = 16
CHUNK = 32
VECS = D_MODEL // LANES


def _sc_body(x_hbm, table_hbm, pos_hbm, out_hbm,
             idx_v, row0, row1, pos0, pos1,
             gsem0, gsem1, psem0, psem1, osem0, osem1, n_idx):
    wid = lax.axis_index("s") * NC + lax.axis_index("c")
    b_per_w = n_idx // NW
    n_chunks = b_per_w // CHUNK
    base = wid * b_per_w
    pos_base = lax.rem(base, MAX_POS)

    rows = [row0, row1]
    poss = [pos0, pos1]
    gsems = [gsem0, gsem1]
    psems = [psem0, psem1]
    osems = [osem0, osem1]

    pltpu.sync_copy(x_hbm.at[pl.ds(base, b_per_w)], idx_v)

    def start_chunk(c):
        b = c % 2
        g = pltpu.async_copy(
            table_hbm.at[idx_v.at[pl.ds(c * CHUNK, CHUNK)]], rows[b], gsems[b])
        p = pltpu.async_copy(
            pos_hbm.at[pl.ds(pos_base + c * CHUNK, CHUNK)], poss[b], psems[b])
        return g, p

    def fma(b):
        rb, pb = rows[b], poss[b]

        def body(i, _):
            for j in range(VECS):
                sl = pl.ds(j * LANES, LANES)
                rb[i, sl] = rb[i, sl] * SCALE + pb[i, sl]
            return 0

        lax.fori_loop(0, CHUNK, body, 0)

    out_copies = [None, None]
    inflight = [start_chunk(0)]
    for c in range(n_chunks):
        b = c % 2
        if c + 1 < n_chunks:
            nb = (c + 1) % 2
            if out_copies[nb] is not None:
                out_copies[nb].wait()
                out_copies[nb] = None
            inflight.append(start_chunk(c + 1))
        g, p = inflight.pop(0)
        g.wait()
        p.wait()
        fma(b)
        out_copies[b] = pltpu.async_copy(
            rows[b], out_hbm.at[pl.ds(base + c * CHUNK, CHUNK)], osems[b])
    for oc in out_copies:
        if oc is not None:
            oc.wait()


def kernel(x, table):
    bsz, length = x.shape
    n_idx = bsz * length
    x_flat = x.reshape(n_idx).astype(jnp.int32)
    pos = jnp.asarray(_POS_ENC)

    mesh = plsc.VectorSubcoreMesh(
        core_axis_name="c", subcore_axis_name="s",
        num_cores=NC, num_subcores=NS)
    sc_call = pl.kernel(
        functools.partial(_sc_body, n_idx=n_idx),
        out_type=jax.ShapeDtypeStruct((n_idx, D_MODEL), jnp.float32),
        mesh=mesh,
        scratch_types=[
            pltpu.VMEM((n_idx // NW,), jnp.int32),
            pltpu.VMEM((CHUNK, D_MODEL), jnp.float32),
            pltpu.VMEM((CHUNK, D_MODEL), jnp.float32),
            pltpu.VMEM((CHUNK, D_MODEL), jnp.float32),
            pltpu.VMEM((CHUNK, D_MODEL), jnp.float32),
            pltpu.SemaphoreType.DMA,
            pltpu.SemaphoreType.DMA,
            pltpu.SemaphoreType.DMA,
            pltpu.SemaphoreType.DMA,
            pltpu.SemaphoreType.DMA,
            pltpu.SemaphoreType.DMA,
        ],
    )
    out = sc_call(x_flat, table, pos)
    return out.reshape(bsz, length, D_MODEL)
